# initial kernel scaffold (unmeasured)
import jax
import jax.numpy as jnp
from jax import lax
from jax.experimental import pallas as pl
from jax.experimental.pallas import tpu as pltpu

M = 8192
D = 2048
NZ = 4
Q = M // NZ


def kernel(partial, resid, gamma):
    p2 = partial.reshape(M, D)
    gamma2 = gamma.reshape(1, D)

    def body(p_ref, resid_ref, gamma_ref, out_ref, rs_ref,
             abuf, bbuf, rs_send, rs_recv, ag_send, ag_recv, lsem):
        ix = lax.axis_index("x")
        iy = lax.axis_index("y")
        iz = lax.axis_index("z")
        left = (ix, iy, (iz + NZ - 1) % NZ)
        right = (ix, iy, (iz + 1) % NZ)

        bar = pltpu.get_barrier_semaphore()
        for nbr in (left, right):
            pl.semaphore_signal(bar, inc=1, device_id=nbr,
                                device_id_type=pl.DeviceIdType.MESH)
        pl.semaphore_wait(bar, 2)

        def copy(src, dst, sem):
            c = pltpu.make_async_copy(src, dst, sem)
            c.start()
            c.wait()

        def rows(q):
            return pl.ds(q * Q, Q)

        for h in range(NZ - 1):
            s_h = (iz + NZ - h) % NZ
            if h == 0:
                src = p_ref.at[rows(s_h), :]
            else:
                copy(rs_ref.at[h - 1], abuf, lsem.at[0])
                copy(p_ref.at[rows(s_h), :], bbuf, lsem.at[1])
                abuf[...] = abuf[...] + bbuf[...]
                src = abuf
            rdma = pltpu.make_async_remote_copy(
                src_ref=src,
                dst_ref=rs_ref.at[h],
                send_sem=rs_send.at[h],
                recv_sem=rs_recv.at[h],
                device_id=right,
                device_id_type=pl.DeviceIdType.MESH,
            )
            rdma.start()
            rdma.wait()

        q_me = (iz + 1) % NZ
        copy(rs_ref.at[NZ - 2], abuf, lsem.at[0])
        copy(p_ref.at[rows(q_me), :], bbuf, lsem.at[1])
        abuf[...] = abuf[...] + bbuf[...]
        copy(resid_ref.at[rows(q_me), :], bbuf, lsem.at[1])
        y = abuf[...] + bbuf[...]
        ms = jnp.mean(y * y, axis=-1, keepdims=True)
        abuf[...] = y * lax.rsqrt(ms + 1e-6) * gamma_ref[...]
        copy(abuf, out_ref.at[rows(q_me), :], lsem.at[2])

        for h in range(NZ - 1):
            a_h = (iz + 1 + NZ - h) % NZ
            rdma = pltpu.make_async_remote_copy(
                src_ref=out_ref.at[rows(a_h), :],
                dst_ref=out_ref.at[rows(a_h), :],
                send_sem=ag_send.at[h],
                recv_sem=ag_recv.at[h],
                device_id=right,
                device_id_type=pl.DeviceIdType.MESH,
            )
            rdma.start()
            rdma.wait()

    out, _ = pl.pallas_call(
        body,
        out_shape=[
            jax.ShapeDtypeStruct((M, D), jnp.float32),
            jax.ShapeDtypeStruct((NZ - 1, Q, D), jnp.float32),
        ],
        in_specs=[
            pl.BlockSpec(memory_space=pltpu.ANY),
            pl.BlockSpec(memory_space=pltpu.ANY),
            pl.BlockSpec(memory_space=pltpu.VMEM),
        ],
        out_specs=[
            pl.BlockSpec(memory_space=pltpu.ANY),
            pl.BlockSpec(memory_space=pltpu.ANY),
        ],
        scratch_shapes=[
            pltpu.VMEM((Q, D), jnp.float32),
            pltpu.VMEM((Q, D), jnp.float32),
            pltpu.SemaphoreType.DMA((NZ - 1,)),
            pltpu.SemaphoreType.DMA((NZ - 1,)),
            pltpu.SemaphoreType.DMA((NZ - 1,)),
            pltpu.SemaphoreType.DMA((NZ - 1,)),
            pltpu.SemaphoreType.DMA((3,)),
        ],
        compiler_params=pltpu.CompilerParams(collective_id=0),
    )(p2, resid, gamma2)
    return out


# baseline (device time: 4410111 ns/iter reference)
import jax
import jax.numpy as jnp
from jax import lax
from jax.experimental import pallas as pl
from jax.experimental.pallas import tpu as pltpu

M = 8192
D = 2048
NZ = 4
R = M // 4


def _specs(n_any):
    return [pl.BlockSpec(memory_space=pl.ANY) for _ in range(n_any)]


def _reduce_call(p_slice, resid_slice, gamma2, dep, leftward, cid):
    first, last = (NZ - 1, 0) if leftward else (0, NZ - 1)

    def body(p_ref, resid_ref, gamma_ref, dep_ref, out_ref,
             abuf, rb, tmp, ssem, rsem, l0, l1):
        ix = lax.axis_index("x")
        iy = lax.axis_index("y")
        iz = lax.axis_index("z")
        for nbr in ((ix, iy, (iz + NZ - 1) % NZ), (ix, iy, (iz + 1) % NZ)):
            pl.semaphore_signal(pltpu.get_barrier_semaphore(), inc=1,
                                device_id=nbr,
                                device_id_type=pl.DeviceIdType.MESH)
        pl.semaphore_wait(pltpu.get_barrier_semaphore(), 2)

        step = -1 if leftward else 1
        tgt = (ix, iy, jnp.clip(iz + step, 0, NZ - 1))

        def copy(src, dst, sem):
            c = pltpu.make_async_copy(src, dst, sem)
            c.start()
            c.wait()

        d = pltpu.make_async_remote_copy(
            src_ref=abuf, dst_ref=rb, send_sem=ssem, recv_sem=rsem,
            device_id=tgt, device_id_type=pl.DeviceIdType.MESH)

        @pl.when(iz == first)
        def _():
            copy(p_ref, abuf, l0)

        @pl.when(iz != first)
        def _():
            d.wait_recv()
            copy(p_ref, abuf, l0)
            for k in range(R // 512):
                rows = pl.ds(k * 512, 512)
                abuf[rows, :] = abuf[rows, :] + rb[rows, :]

        @pl.when(iz != last)
        def _():
            d.start()
            d.wait_send()

        @pl.when(iz == last)
        def _():
            for k in range(R // 512):
                rows = pl.ds(k * 512, 512)
                copy(resid_ref.at[rows, :], tmp, l0)
                y = abuf[rows, :] + tmp[...]
                ms = jnp.mean(y * y, axis=-1, keepdims=True)
                abuf[rows, :] = y * lax.rsqrt(ms + 1e-6) * gamma_ref[...]

        copy(abuf, out_ref, l1)

    return pl.pallas_call(
        body,
        out_shape=jax.ShapeDtypeStruct((R, D), jnp.float32),
        in_specs=_specs(2) + [pl.BlockSpec(memory_space=pltpu.VMEM)]
        + _specs(1),
        out_specs=pl.BlockSpec(memory_space=pl.ANY),
        scratch_shapes=[
            pltpu.VMEM((R, D), jnp.float32),
            pltpu.VMEM((R, D), jnp.float32),
            pltpu.VMEM((512, D), jnp.float32),
            pltpu.SemaphoreType.DMA, pltpu.SemaphoreType.DMA,
            pltpu.SemaphoreType.DMA, pltpu.SemaphoreType.DMA,
        ],
        compiler_params=pltpu.CompilerParams(
            collective_id=cid, vmem_limit_bytes=56 * 1024 * 1024),
    )(p_slice, resid_slice, gamma2, dep)


def _bcast_call(x_slice, dep, leftward, cid):
    src_dev, snk_dev = (NZ - 1, 0) if leftward else (0, NZ - 1)

    def body(x_ref, dep_ref, out_ref, abuf, rb, ssem, rsem, l0):
        ix = lax.axis_index("x")
        iy = lax.axis_index("y")
        iz = lax.axis_index("z")
        for nbr in ((ix, iy, (iz + NZ - 1) % NZ), (ix, iy, (iz + 1) % NZ)):
            pl.semaphore_signal(pltpu.get_barrier_semaphore(), inc=1,
                                device_id=nbr,
                                device_id_type=pl.DeviceIdType.MESH)
        pl.semaphore_wait(pltpu.get_barrier_semaphore(), 2)

        step = -1 if leftward else 1
        tgt = (ix, iy, jnp.clip(iz + step, 0, NZ - 1))

        def copy(src, dst, sem):
            c = pltpu.make_async_copy(src, dst, sem)
            c.start()
            c.wait()

        dsrc = pltpu.make_async_remote_copy(
            src_ref=abuf, dst_ref=rb, send_sem=ssem, recv_sem=rsem,
            device_id=tgt, device_id_type=pl.DeviceIdType.MESH)
        dfwd = pltpu.make_async_remote_copy(
            src_ref=rb, dst_ref=rb, send_sem=ssem, recv_sem=rsem,
            device_id=tgt, device_id_type=pl.DeviceIdType.MESH)

        @pl.when(iz == src_dev)
        def _():
            copy(x_ref, abuf, l0)
            dsrc.start()
            dsrc.wait_send()
            copy(abuf, out_ref, l0)

        @pl.when(iz != src_dev)
        def _():
            dfwd.wait_recv()
            copy(rb, out_ref, l0)

        @pl.when(jnp.logical_and(iz != src_dev, iz != snk_dev))
        def _():
            dfwd.start()
            dfwd.wait_send()

    return pl.pallas_call(
        body,
        out_shape=jax.ShapeDtypeStruct((R, D), jnp.float32),
        in_specs=_specs(2),
        out_specs=pl.BlockSpec(memory_space=pl.ANY),
        scratch_shapes=[
            pltpu.VMEM((R, D), jnp.float32),
            pltpu.VMEM((R, D), jnp.float32),
            pltpu.SemaphoreType.DMA, pltpu.SemaphoreType.DMA,
            pltpu.SemaphoreType.DMA,
        ],
        compiler_params=pltpu.CompilerParams(
            collective_id=cid, vmem_limit_bytes=56 * 1024 * 1024),
    )(x_slice, dep)


def kernel(partial, resid, gamma):
    p2 = partial.reshape(M, D)
    gamma2 = gamma.reshape(1, D)

    a1 = _reduce_call(p2[0:R], resid[0:R], gamma2, gamma2, False, 0)
    a2 = _reduce_call(p2[R:2 * R], resid[R:2 * R], gamma2, a1, False, 1)
    b1 = _reduce_call(p2[2 * R:3 * R], resid[2 * R:3 * R], gamma2, a2,
                      True, 2)
    b2 = _reduce_call(p2[3 * R:4 * R], resid[3 * R:4 * R], gamma2, b1,
                      True, 3)
    oa1 = _bcast_call(a1, b2, True, 4)
    oa2 = _bcast_call(a2, oa1, True, 5)
    ob1 = _bcast_call(b1, oa2, False, 6)
    ob2 = _bcast_call(b2, ob1, False, 7)
    return jnp.concatenate([oa1, oa2, ob1, ob2], axis=0)
